# baseline (speedup 1.0000x reference)
import functools

import jax
import jax.numpy as jnp
from jax.experimental import pallas as pl
from jax.experimental.pallas import tpu as pltpu


def _block(na_i, gb_i, h_i,
           obs_ref, wcat_ref, bcat_ref, p_ref, q_ref,
           wg1_ref, bg1_ref, wg2_ref, bg2_ref,
           wpost_ref, bpost_ref,
           w1t_ref, w1b_ref, b1_ref, w2_ref, b2_ref, w3_ref, b3_ref,
           out_ref):
    f32 = jnp.float32
    na = na_i
    gb = gb_i
    h = h_i
    r = gb * na

    def mm(a, b):
        return jnp.dot(a, b, preferred_element_type=f32)

    obs = obs_ref[...].reshape(r, obs_ref.shape[2])
    y = jnp.maximum(mm(obs, wcat_ref[...]) + bcat_ref[...], 0.0)
    g = y[:, :h]
    lo = y[:, h:]

    mg = mm(p_ref[...], g)
    x1 = jnp.maximum(mm(mg, wg1_ref[...]) + bg1_ref[...], 0.0)
    x2 = jnp.maximum(mm(x1, wg2_ref[...]) + bg2_ref[...], 0.0)
    go = jnp.maximum(mm(x2, wpost_ref[...]) + bpost_ref[...], 0.0)

    u = mm(go, w1t_ref[...])
    h1 = jnp.maximum(mm(q_ref[...], u) + mm(lo, w1b_ref[...]) + b1_ref[...],
                     0.0)
    h2 = jnp.maximum(mm(h1, w2_ref[...]) + b2_ref[...], 0.0)
    q = mm(h2, w3_ref[...]) + b3_ref[...]
    out_ref[...] = q.reshape(gb, na, q.shape[1])


def kernel(obs_j, W_pre, b_pre, W_g1, b_g1, W_g2, b_g2, W_post, b_post,
           W_loc, b_loc, W1, b1, W2, b2, W3, b3):
    B, NA, OBS = obs_j.shape
    H = W_pre.shape[1]
    GE = W_post.shape[1]
    LE = W_loc.shape[1]
    F1 = W1.shape[1]
    F2 = W2.shape[1]
    NACT = W3.shape[1]

    GB = 128
    while B % GB:
        GB //= 2
    R = GB * NA

    Wcat = jnp.concatenate([W_pre, W_loc], axis=1)
    bcat = jnp.concatenate([b_pre, b_loc]).reshape(1, -1)
    W1t = W1[:GE]
    W1b = W1[GE:]

    gid = jnp.arange(R, dtype=jnp.int32) // NA
    gsel = jnp.arange(GB, dtype=jnp.int32)
    P = (gsel[:, None] == gid[None, :]).astype(jnp.float32) * (1.0 / NA)
    Q = (gid[:, None] == gsel[None, :]).astype(jnp.float32)

    def b2d(v):
        return v.reshape(1, -1)

    full = lambda shp: pl.BlockSpec(shp, lambda i: (0, 0))
    kern = functools.partial(_block, NA, GB, H)

    out = pl.pallas_call(
        kern,
        grid=(B // GB,),
        in_specs=[
            pl.BlockSpec((GB, NA, OBS), lambda i: (i, 0, 0)),
            full((OBS, H + LE)), full((1, H + LE)),
            full((GB, R)), full((R, GB)),
            full((H, H)), full((1, H)),
            full((H, H)), full((1, H)),
            full((H, GE)), full((1, GE)),
            full((GE, F1)), full((LE, F1)), full((1, F1)),
            full((F1, F2)), full((1, F2)),
            full((F2, NACT)), full((1, NACT)),
        ],
        out_specs=pl.BlockSpec((GB, NA, NACT), lambda i: (i, 0, 0)),
        out_shape=jax.ShapeDtypeStruct((B, NA, NACT), jnp.float32),
        compiler_params=pltpu.CompilerParams(
            dimension_semantics=("parallel",),
        ),
    )(obs_j, Wcat, bcat, P, Q,
      W_g1, b2d(b_g1), W_g2, b2d(b_g2),
      W_post, b2d(b_post),
      W1t, W1b, b2d(b1), W2, b2d(b2), W3, b2d(b3))

    return out


# speedup vs baseline: 1.0794x; 1.0794x over previous
import functools

import jax
import jax.numpy as jnp
from jax.experimental import pallas as pl
from jax.experimental.pallas import tpu as pltpu


def _block(na_i, gb_i, h_i,
           obs_ref, wcat_ref, bcat_ref,
           wg1_ref, bg1_ref, wg2_ref, bg2_ref,
           wpost_ref, bpost_ref,
           w1t_ref, w1b_ref, b1_ref, w2_ref, b2_ref, w3_ref, b3_ref,
           out_ref):
    f32 = jnp.float32
    na = na_i
    gb = gb_i
    h = h_i
    r = gb * na

    def mm(a, b):
        return jnp.dot(a, b, preferred_element_type=f32)

    obs = obs_ref[...].reshape(r, obs_ref.shape[2])
    y = jnp.maximum(mm(obs, wcat_ref[...]) + bcat_ref[...], 0.0)
    g = y[:, :h]
    lo = y[:, h:]

    prow = jax.lax.broadcasted_iota(jnp.int32, (gb, r), 0)
    pcol = jax.lax.broadcasted_iota(jnp.int32, (gb, r), 1)
    P = jnp.where(pcol // na == prow, f32(1.0 / na), f32(0.0))
    mg = mm(P, g)
    x1 = jnp.maximum(mm(mg, wg1_ref[...]) + bg1_ref[...], 0.0)
    x2 = jnp.maximum(mm(x1, wg2_ref[...]) + bg2_ref[...], 0.0)
    go = jnp.maximum(mm(x2, wpost_ref[...]) + bpost_ref[...], 0.0)

    u = mm(go, w1t_ref[...])
    qrow = jax.lax.broadcasted_iota(jnp.int32, (r, gb), 0)
    qcol = jax.lax.broadcasted_iota(jnp.int32, (r, gb), 1)
    Q = jnp.where(qrow // na == qcol, f32(1.0), f32(0.0))
    h1 = jnp.maximum(mm(Q, u) + mm(lo, w1b_ref[...]) + b1_ref[...],
                     0.0)
    h2 = jnp.maximum(mm(h1, w2_ref[...]) + b2_ref[...], 0.0)
    q = mm(h2, w3_ref[...]) + b3_ref[...]
    out_ref[...] = q.reshape(gb, na, q.shape[1])


def kernel(obs_j, W_pre, b_pre, W_g1, b_g1, W_g2, b_g2, W_post, b_post,
           W_loc, b_loc, W1, b1, W2, b2, W3, b3):
    B, NA, OBS = obs_j.shape
    H = W_pre.shape[1]
    GE = W_post.shape[1]
    LE = W_loc.shape[1]
    F1 = W1.shape[1]
    F2 = W2.shape[1]
    NACT = W3.shape[1]

    GB = 128
    while B % GB:
        GB //= 2
    R = GB * NA

    Wcat = jnp.concatenate([W_pre, W_loc], axis=1)
    bcat = jnp.concatenate([b_pre, b_loc]).reshape(1, -1)
    W1t = W1[:GE]
    W1b = W1[GE:]

    def b2d(v):
        return v.reshape(1, -1)

    full = lambda shp: pl.BlockSpec(shp, lambda i: (0, 0))
    kern = functools.partial(_block, NA, GB, H)

    out = pl.pallas_call(
        kern,
        grid=(B // GB,),
        in_specs=[
            pl.BlockSpec((GB, NA, OBS), lambda i: (i, 0, 0)),
            full((OBS, H + LE)), full((1, H + LE)),
            full((H, H)), full((1, H)),
            full((H, H)), full((1, H)),
            full((H, GE)), full((1, GE)),
            full((GE, F1)), full((LE, F1)), full((1, F1)),
            full((F1, F2)), full((1, F2)),
            full((F2, NACT)), full((1, NACT)),
        ],
        out_specs=pl.BlockSpec((GB, NA, NACT), lambda i: (i, 0, 0)),
        out_shape=jax.ShapeDtypeStruct((B, NA, NACT), jnp.float32),
        compiler_params=pltpu.CompilerParams(
            dimension_semantics=("parallel",),
        ),
    )(obs_j, Wcat, bcat,
      W_g1, b2d(b_g1), W_g2, b2d(b_g2),
      W_post, b2d(b_post),
      W1t, W1b, b2d(b1), W2, b2d(b2), W3, b2d(b3))

    return out


# speedup vs baseline: 1.1519x; 1.0672x over previous
import functools

import jax
import jax.numpy as jnp
from jax.experimental import pallas as pl
from jax.experimental.pallas import tpu as pltpu


def _block(na_i, gb_i, h_i,
           obs_ref, wpre_ref, bpre_ref, wloc_ref, bloc_ref,
           wg1_ref, bg1_ref, wg2_ref, bg2_ref,
           wpost_ref, bpost_ref,
           w1t_ref, w1b_ref, b1_ref, w2_ref, b2_ref, w3_ref, b3_ref,
           out_ref):
    f32 = jnp.float32
    na = na_i
    gb = gb_i
    h = h_i
    r = gb * na

    def mm(a, b):
        return jnp.dot(a, b, preferred_element_type=f32)

    obs = obs_ref[...].reshape(r, obs_ref.shape[2])
    g = jnp.maximum(mm(obs, wpre_ref[...]) + bpre_ref[...], 0.0)
    lo = jnp.maximum(mm(obs, wloc_ref[...]) + bloc_ref[...], 0.0)

    prow = jax.lax.broadcasted_iota(jnp.int32, (gb, r), 0)
    pcol = jax.lax.broadcasted_iota(jnp.int32, (gb, r), 1)
    P = jnp.where(pcol // na == prow, f32(1.0 / na), f32(0.0))
    mg = mm(P, g)
    x1 = jnp.maximum(mm(mg, wg1_ref[...]) + bg1_ref[...], 0.0)
    x2 = jnp.maximum(mm(x1, wg2_ref[...]) + bg2_ref[...], 0.0)
    go = jnp.maximum(mm(x2, wpost_ref[...]) + bpost_ref[...], 0.0)

    u = mm(go, w1t_ref[...])
    qrow = jax.lax.broadcasted_iota(jnp.int32, (r, gb), 0)
    qcol = jax.lax.broadcasted_iota(jnp.int32, (r, gb), 1)
    Q = jnp.where(qrow // na == qcol, f32(1.0), f32(0.0))
    h1 = jnp.maximum(mm(Q, u) + mm(lo, w1b_ref[...]) + b1_ref[...],
                     0.0)
    h2 = jnp.maximum(mm(h1, w2_ref[...]) + b2_ref[...], 0.0)
    q = mm(h2, w3_ref[...]) + b3_ref[...]
    out_ref[...] = q.reshape(gb, na, q.shape[1])


def kernel(obs_j, W_pre, b_pre, W_g1, b_g1, W_g2, b_g2, W_post, b_post,
           W_loc, b_loc, W1, b1, W2, b2, W3, b3):
    B, NA, OBS = obs_j.shape
    H = W_pre.shape[1]
    GE = W_post.shape[1]
    LE = W_loc.shape[1]
    F1 = W1.shape[1]
    F2 = W2.shape[1]
    NACT = W3.shape[1]

    GB = 128
    while B % GB:
        GB //= 2
    R = GB * NA

    W1t = W1[:GE]
    W1b = W1[GE:]

    def b2d(v):
        return v.reshape(1, -1)

    full = lambda shp: pl.BlockSpec(shp, lambda i: (0, 0))
    kern = functools.partial(_block, NA, GB, H)

    out = pl.pallas_call(
        kern,
        grid=(B // GB,),
        in_specs=[
            pl.BlockSpec((GB, NA, OBS), lambda i: (i, 0, 0)),
            full((OBS, H)), full((1, H)),
            full((OBS, LE)), full((1, LE)),
            full((H, H)), full((1, H)),
            full((H, H)), full((1, H)),
            full((H, GE)), full((1, GE)),
            full((GE, F1)), full((LE, F1)), full((1, F1)),
            full((F1, F2)), full((1, F2)),
            full((F2, NACT)), full((1, NACT)),
        ],
        out_specs=pl.BlockSpec((GB, NA, NACT), lambda i: (i, 0, 0)),
        out_shape=jax.ShapeDtypeStruct((B, NA, NACT), jnp.float32),
        compiler_params=pltpu.CompilerParams(
            dimension_semantics=("parallel",),
        ),
    )(obs_j, W_pre, b2d(b_pre), W_loc, b2d(b_loc),
      W_g1, b2d(b_g1), W_g2, b2d(b_g2),
      W_post, b2d(b_post),
      W1t, W1b, b2d(b1), W2, b2d(b2), W3, b2d(b3))

    return out


# speedup vs baseline: 1.1523x; 1.0004x over previous
import functools

import jax
import jax.numpy as jnp
from jax.experimental import pallas as pl
from jax.experimental.pallas import tpu as pltpu


def _block(na_i, gb_i, h_i,
           obs_ref, wpre_ref, bpre_ref, wloc_ref, bloc_ref,
           wg1_ref, bg1_ref, wg2_ref, bg2_ref,
           wpost_ref, bpost_ref,
           w1t_ref, w1b_ref, b1_ref, w2_ref, b2_ref, w3_ref, b3_ref,
           out_ref):
    f32 = jnp.float32
    na = na_i
    gb = gb_i
    h = h_i
    r = gb * na

    def mm(a, b):
        return jnp.dot(a, b, preferred_element_type=f32)

    obs = obs_ref[...].reshape(r, obs_ref.shape[2])
    g = jnp.maximum(mm(obs, wpre_ref[...]) + bpre_ref[...], 0.0)
    lo = jnp.maximum(mm(obs, wloc_ref[...]) + bloc_ref[...], 0.0)

    prow = jax.lax.broadcasted_iota(jnp.int32, (gb, r), 0)
    pcol = jax.lax.broadcasted_iota(jnp.int32, (gb, r), 1)
    P = jnp.where(pcol // na == prow, f32(1.0 / na), f32(0.0))
    mg = mm(P, g)
    x1 = jnp.maximum(mm(mg, wg1_ref[...]) + bg1_ref[...], 0.0)
    x2 = jnp.maximum(mm(x1, wg2_ref[...]) + bg2_ref[...], 0.0)
    go = jnp.maximum(mm(x2, wpost_ref[...]) + bpost_ref[...], 0.0)

    u = mm(go, w1t_ref[...])
    qrow = jax.lax.broadcasted_iota(jnp.int32, (r, gb), 0)
    qcol = jax.lax.broadcasted_iota(jnp.int32, (r, gb), 1)
    Q = jnp.where(qrow // na == qcol, f32(1.0), f32(0.0))
    h1 = jnp.maximum(mm(Q, u) + mm(lo, w1b_ref[...]) + b1_ref[...],
                     0.0)
    h2 = jnp.maximum(mm(h1, w2_ref[...]) + b2_ref[...], 0.0)
    q = mm(h2, w3_ref[...]) + b3_ref[...]
    out_ref[...] = q.reshape(gb, na, q.shape[1])


def kernel(obs_j, W_pre, b_pre, W_g1, b_g1, W_g2, b_g2, W_post, b_post,
           W_loc, b_loc, W1, b1, W2, b2, W3, b3):
    B, NA, OBS = obs_j.shape
    H = W_pre.shape[1]
    GE = W_post.shape[1]
    LE = W_loc.shape[1]
    F1 = W1.shape[1]
    F2 = W2.shape[1]
    NACT = W3.shape[1]

    GB = 256
    while B % GB:
        GB //= 2
    R = GB * NA

    W1t = W1[:GE]
    W1b = W1[GE:]

    def b2d(v):
        return v.reshape(1, -1)

    full = lambda shp: pl.BlockSpec(shp, lambda i: (0, 0))
    kern = functools.partial(_block, NA, GB, H)

    out = pl.pallas_call(
        kern,
        grid=(B // GB,),
        in_specs=[
            pl.BlockSpec((GB, NA, OBS), lambda i: (i, 0, 0)),
            full((OBS, H)), full((1, H)),
            full((OBS, LE)), full((1, LE)),
            full((H, H)), full((1, H)),
            full((H, H)), full((1, H)),
            full((H, GE)), full((1, GE)),
            full((GE, F1)), full((LE, F1)), full((1, F1)),
            full((F1, F2)), full((1, F2)),
            full((F2, NACT)), full((1, NACT)),
        ],
        out_specs=pl.BlockSpec((GB, NA, NACT), lambda i: (i, 0, 0)),
        out_shape=jax.ShapeDtypeStruct((B, NA, NACT), jnp.float32),
        compiler_params=pltpu.CompilerParams(
            dimension_semantics=("parallel",),
        ),
    )(obs_j, W_pre, b2d(b_pre), W_loc, b2d(b_loc),
      W_g1, b2d(b_g1), W_g2, b2d(b_g2),
      W_post, b2d(b_post),
      W1t, W1b, b2d(b1), W2, b2d(b2), W3, b2d(b3))

    return out


# speedup vs baseline: 1.4534x; 1.2613x over previous
import functools

import jax
import jax.numpy as jnp
from jax.experimental import pallas as pl
from jax.experimental.pallas import tpu as pltpu


def _block(na_i, gb_i, h_i,
           obs_ref, wpre_ref, bpre_ref, wloc_ref, bloc_ref,
           wg1_ref, bg1_ref, wg2_ref, bg2_ref,
           wpost_ref, bpost_ref,
           w1t_ref, w1b_ref, b1_ref, w2_ref, b2_ref, w3_ref, b3_ref,
           out_ref):
    f32 = jnp.float32
    na = na_i
    gb = gb_i
    h = h_i
    r = gb * na

    def mm(a, b):
        return jnp.dot(a, b, preferred_element_type=f32)

    obs = obs_ref[...].reshape(r, obs_ref.shape[2])
    g = jnp.maximum(mm(obs, wpre_ref[...]) + bpre_ref[...], 0.0)
    lo = jnp.maximum(mm(obs, wloc_ref[...]) + bloc_ref[...], 0.0)

    mg = jnp.sum(g.reshape(gb, na, h), axis=1) * f32(1.0 / na)
    x1 = jnp.maximum(mm(mg, wg1_ref[...]) + bg1_ref[...], 0.0)
    x2 = jnp.maximum(mm(x1, wg2_ref[...]) + bg2_ref[...], 0.0)
    go = jnp.maximum(mm(x2, wpost_ref[...]) + bpost_ref[...], 0.0)

    u = mm(go, w1t_ref[...])
    f1 = u.shape[1]
    ub = jnp.broadcast_to(u[:, None, :], (gb, na, f1)).reshape(r, f1)
    h1 = jnp.maximum(ub + mm(lo, w1b_ref[...]) + b1_ref[...],
                     0.0)
    h2 = jnp.maximum(mm(h1, w2_ref[...]) + b2_ref[...], 0.0)
    q = mm(h2, w3_ref[...]) + b3_ref[...]
    out_ref[...] = q.reshape(gb, na, q.shape[1])


def kernel(obs_j, W_pre, b_pre, W_g1, b_g1, W_g2, b_g2, W_post, b_post,
           W_loc, b_loc, W1, b1, W2, b2, W3, b3):
    B, NA, OBS = obs_j.shape
    H = W_pre.shape[1]
    GE = W_post.shape[1]
    LE = W_loc.shape[1]
    F1 = W1.shape[1]
    F2 = W2.shape[1]
    NACT = W3.shape[1]

    GB = 256
    while B % GB:
        GB //= 2
    R = GB * NA

    W1t = W1[:GE]
    W1b = W1[GE:]

    def b2d(v):
        return v.reshape(1, -1)

    full = lambda shp: pl.BlockSpec(shp, lambda i: (0, 0))
    kern = functools.partial(_block, NA, GB, H)

    out = pl.pallas_call(
        kern,
        grid=(B // GB,),
        in_specs=[
            pl.BlockSpec((GB, NA, OBS), lambda i: (i, 0, 0)),
            full((OBS, H)), full((1, H)),
            full((OBS, LE)), full((1, LE)),
            full((H, H)), full((1, H)),
            full((H, H)), full((1, H)),
            full((H, GE)), full((1, GE)),
            full((GE, F1)), full((LE, F1)), full((1, F1)),
            full((F1, F2)), full((1, F2)),
            full((F2, NACT)), full((1, NACT)),
        ],
        out_specs=pl.BlockSpec((GB, NA, NACT), lambda i: (i, 0, 0)),
        out_shape=jax.ShapeDtypeStruct((B, NA, NACT), jnp.float32),
        compiler_params=pltpu.CompilerParams(
            dimension_semantics=("parallel",),
        ),
    )(obs_j, W_pre, b2d(b_pre), W_loc, b2d(b_loc),
      W_g1, b2d(b_g1), W_g2, b2d(b_g2),
      W_post, b2d(b_post),
      W1t, W1b, b2d(b1), W2, b2d(b2), W3, b2d(b3))

    return out
